# baseline (speedup 1.0000x reference)
import functools

import jax
import jax.numpy as jnp
from jax import lax
from jax.experimental import pallas as pl
from jax.experimental.pallas import tpu as pltpu
from jax.experimental.pallas import tpu_sc as plsc

_L = 16


@functools.partial(jax.jit, static_argnums=(3, 4, 5, 6))
def _sc_lookup(pair_tensor, lut, emb_flat, Bo, N, D, W):
    info = plsc.get_sparse_core_info()
    NW = info.num_cores * info.num_subcores
    B = Bo * N
    per_w = B // NW
    ro_w = Bo // NW
    RPC = 2
    CH = RPC * N
    n_ch = ro_w // RPC
    mesh = plsc.VectorSubcoreMesh(core_axis_name="c", subcore_axis_name="s")

    @functools.partial(
        pl.kernel,
        mesh=mesh,
        compiler_params=pltpu.CompilerParams(
            use_tc_tiling_on_sc=False, needs_layout_passes=False),
        out_type=jax.ShapeDtypeStruct((B * D,), jnp.float32),
        scratch_types=[
            pltpu.VMEM((RPC, N, 2), jnp.int32),
            pltpu.VMEM((W * W,), jnp.int32),
            pltpu.VMEM((D * D,), jnp.float32),
            pltpu.VMEM((CH * D,), jnp.float32),
            pltpu.VMEM((CH * D,), jnp.float32),
            pltpu.SemaphoreType.DMA,
        ],
    )
    def body(pair_hbm, lut_hbm, emb_hbm, out_hbm,
             pair_v, lut_v, emb_v, rows0, rows1, wsem):
        wid = lax.axis_index("s") * info.num_cores + lax.axis_index("c")
        base = wid * per_w
        pltpu.sync_copy(lut_hbm, lut_v)
        pltpu.sync_copy(emb_hbm, emb_v)

        rows = (rows0, rows1)
        col = jnp.arange(_L, dtype=jnp.int32)

        def start_write(c, buf):
            pltpu.async_copy(
                buf, out_hbm.at[pl.ds((base + c * CH) * D, CH * D)], wsem)

        def drain_write(buf):
            pltpu.make_async_copy(
                buf, out_hbm.at[pl.ds(base * D, CH * D)], wsem).wait()

        def expand_chunk(c, buf):
            pltpu.sync_copy(pair_hbm.at[pl.ds(wid * ro_w + c * RPC, RPC)],
                            pair_v)

            def group(g, carry):
                q = g * _L + col
                qr = q // N
                qn = q % N
                av = plsc.load_gather(pair_v, [qr, qn, col * 0])
                bv = plsc.load_gather(pair_v, [qr, qn, col * 0 + 1])
                pos = av * W + bv
                off = plsc.load_gather(lut_v, [pos]) * D
                dst0 = g * _L * D
                pend = []
                for i in range(_L):
                    addrs = lax.gather(
                        off, jnp.full((_L, 1), i, jnp.int32),
                        lax.GatherDimensionNumbers(
                            offset_dims=(), collapsed_slice_dims=(0,),
                            start_index_map=(0,)),
                        slice_sizes=(1,),
                        mode=lax.GatherScatterMode.PROMISE_IN_BOUNDS) + col
                    vals = [plsc.load_gather(emb_v, [addrs + j * _L])
                            for j in range(D // _L)]
                    if len(pend) == 2:
                        pdst, pvals = pend.pop(0)
                        for j, v in enumerate(pvals):
                            buf[pl.ds(pdst + j * _L, _L)] = v
                    pend.append((dst0 + i * D, vals))
                for pdst, pvals in pend:
                    for j, v in enumerate(pvals):
                        buf[pl.ds(pdst + j * _L, _L)] = v
                return carry

            lax.fori_loop(0, CH // _L, group, 0, unroll=2)

        def g_body(g, carry):
            for sub in range(2):
                c = g * 2 + sub
                buf = rows[sub]

                @pl.when(c >= 2)
                def _():
                    drain_write(buf)

                expand_chunk(c, buf)
                start_write(c, buf)
            return carry

        lax.fori_loop(0, n_ch // 2, g_body, 0)
        drain_write(rows0)
        drain_write(rows1)

    return body(pair_tensor, lut, emb_flat)


def kernel(pair_tensor, lookup_table, embedding):
    Bo, N, _ = pair_tensor.shape
    D = embedding.shape[1]
    W = lookup_table.shape[1]
    lut = lookup_table.reshape(W * W)
    out = _sc_lookup(pair_tensor, lut, embedding.reshape(D * D),
                     Bo, N, D, W)
    return out.reshape(Bo, N, D)


# speedup vs baseline: 3.0069x; 3.0069x over previous
import functools

import jax
import jax.numpy as jnp
from jax import lax
from jax.experimental import pallas as pl
from jax.experimental.pallas import tpu as pltpu
from jax.experimental.pallas import tpu_sc as plsc

_L = 16


@functools.partial(jax.jit, static_argnums=(3, 4, 5))
def _sc_lookup(pos2d, lut, emb_flat, N, D, W):
    info = plsc.get_sparse_core_info()
    NW = info.num_cores * info.num_subcores
    Bo = pos2d.shape[0]
    B = Bo * N
    per_w = B // NW
    ro_w = Bo // NW
    CH = 512
    n_ch = per_w // CH
    mesh = plsc.VectorSubcoreMesh(core_axis_name="c", subcore_axis_name="s")

    @functools.partial(
        pl.kernel,
        mesh=mesh,
        compiler_params=pltpu.CompilerParams(needs_layout_passes=False),
        out_type=jax.ShapeDtypeStruct((B * D,), jnp.float32),
        scratch_types=[
            pltpu.VMEM((ro_w, N), jnp.int32),
            pltpu.VMEM((W * W,), jnp.int32),
            pltpu.VMEM((D * D,), jnp.float32),
            pltpu.VMEM((CH * D,), jnp.float32),
            pltpu.VMEM((CH * D,), jnp.float32),
            pltpu.SemaphoreType.DMA,
        ],
    )
    def body(pos_hbm, lut_hbm, emb_hbm, out_hbm,
             pos_v, lut_v, emb_v, rows0, rows1, wsem):
        wid = lax.axis_index("s") * info.num_cores + lax.axis_index("c")
        base = wid * per_w
        pltpu.sync_copy(lut_hbm, lut_v)
        pltpu.sync_copy(emb_hbm, emb_v)
        pltpu.sync_copy(pos_hbm.at[pl.ds(wid * ro_w, ro_w)], pos_v)

        rows = (rows0, rows1)
        col = jnp.arange(_L, dtype=jnp.int32)

        def start_write(c, buf):
            pltpu.async_copy(
                buf, out_hbm.at[pl.ds((base + c * CH) * D, CH * D)], wsem)

        def drain_write(buf):
            pltpu.make_async_copy(
                buf, out_hbm.at[pl.ds(base * D, CH * D)], wsem).wait()

        def expand_chunk(c, buf):
            r0 = c * CH

            def group(g, carry):
                q = r0 + g * _L + col
                pos = plsc.load_gather(pos_v, [q // N, q % N])
                off = plsc.load_gather(lut_v, [pos]) * D
                dst0 = g * _L * D
                pend = []
                for i in range(_L):
                    addrs = lax.gather(
                        off, jnp.full((_L, 1), i, jnp.int32),
                        lax.GatherDimensionNumbers(
                            offset_dims=(), collapsed_slice_dims=(0,),
                            start_index_map=(0,)),
                        slice_sizes=(1,),
                        mode=lax.GatherScatterMode.PROMISE_IN_BOUNDS) + col
                    vals = [plsc.load_gather(emb_v, [addrs + j * _L])
                            for j in range(D // _L)]
                    if len(pend) == 2:
                        pdst, pvals = pend.pop(0)
                        for j, v in enumerate(pvals):
                            buf[pl.ds(pdst + j * _L, _L)] = v
                    pend.append((dst0 + i * D, vals))
                for pdst, pvals in pend:
                    for j, v in enumerate(pvals):
                        buf[pl.ds(pdst + j * _L, _L)] = v
                return carry

            lax.fori_loop(0, CH // _L, group, 0, unroll=2)

        def g_body(g, carry):
            for sub in range(2):
                c = g * 2 + sub
                buf = rows[sub]

                @pl.when(c >= 2)
                def _():
                    drain_write(buf)

                expand_chunk(c, buf)
                start_write(c, buf)
            return carry

        lax.fori_loop(0, n_ch // 2, g_body, 0)
        drain_write(rows0)
        drain_write(rows1)

    return body(pos2d, lut, emb_flat)


def kernel(pair_tensor, lookup_table, embedding):
    Bo, N, _ = pair_tensor.shape
    D = embedding.shape[1]
    W = lookup_table.shape[1]
    pos2d = pair_tensor[..., 0] * W + pair_tensor[..., 1]
    lut = lookup_table.reshape(W * W)
    out = _sc_lookup(pos2d, lut, embedding.reshape(D * D), N, D, W)
    return out.reshape(Bo, N, D)


# speedup vs baseline: 5.5314x; 1.8396x over previous
import functools

import jax
import jax.numpy as jnp
from jax import lax
from jax.experimental import pallas as pl
from jax.experimental.pallas import tpu as pltpu
from jax.experimental.pallas import tpu_sc as plsc

_L = 16


@functools.partial(jax.jit, static_argnums=(3, 4, 5))
def _sc_lookup(pos2d, lut, emb_flat, N, D, W):
    info = plsc.get_sparse_core_info()
    NW = info.num_cores * info.num_subcores
    Bo = pos2d.shape[0]
    B = Bo * N
    per_w = B // NW
    ro_w = Bo // NW
    CH = 256
    n_ch = per_w // CH
    mesh = plsc.VectorSubcoreMesh(core_axis_name="c", subcore_axis_name="s")

    @functools.partial(
        pl.kernel,
        mesh=mesh,
        compiler_params=pltpu.CompilerParams(needs_layout_passes=False),
        out_type=jax.ShapeDtypeStruct((B, D), jnp.float32),
        scratch_types=[
            pltpu.VMEM((ro_w, N), jnp.int32),
            pltpu.VMEM((W * W,), jnp.int32),
            pltpu.VMEM((D * D,), jnp.float32),
            pltpu.VMEM((CH, D), jnp.float32),
            pltpu.VMEM((CH, D), jnp.float32),
            pltpu.SemaphoreType.DMA,
        ],
    )
    def body(pos_hbm, lut_hbm, emb_hbm, out_hbm,
             pos_v, lut_v, emb_v, rows0, rows1, wsem):
        wid = lax.axis_index("s") * info.num_cores + lax.axis_index("c")
        base = wid * per_w
        pltpu.sync_copy(lut_hbm, lut_v)
        pltpu.sync_copy(emb_hbm, emb_v)
        pltpu.sync_copy(pos_hbm.at[pl.ds(wid * ro_w, ro_w)], pos_v)

        rows = (rows0, rows1)
        col = jnp.arange(_L, dtype=jnp.int32)

        def start_write(c, buf):
            pltpu.async_copy(
                buf, out_hbm.at[pl.ds(base + c * CH, CH)], wsem)

        def drain_write(buf):
            pltpu.make_async_copy(
                buf, out_hbm.at[pl.ds(base, CH)], wsem).wait()

        def expand_chunk(c, buf):
            r0 = c * CH

            def group(g, carry):
                q = r0 + g * _L + col
                pos = plsc.load_gather(pos_v, [q // N, q % N])
                off = plsc.load_gather(lut_v, [pos]) * D
                dst0 = g * _L
                pend = []
                for i in range(_L):
                    addrs = lax.gather(
                        off, jnp.full((_L, 1), i, jnp.int32),
                        lax.GatherDimensionNumbers(
                            offset_dims=(), collapsed_slice_dims=(0,),
                            start_index_map=(0,)),
                        slice_sizes=(1,),
                        mode=lax.GatherScatterMode.PROMISE_IN_BOUNDS) + col
                    vals = [plsc.load_gather(emb_v, [addrs + j * _L])
                            for j in range(D // _L)]
                    if len(pend) == 2:
                        pdst, pvals = pend.pop(0)
                        for j, v in enumerate(pvals):
                            buf[pdst, pl.ds(j * _L, _L)] = v
                    pend.append((dst0 + i, vals))
                for pdst, pvals in pend:
                    for j, v in enumerate(pvals):
                        buf[pdst, pl.ds(j * _L, _L)] = v
                return carry

            lax.fori_loop(0, CH // _L, group, 0, unroll=2)

        def g_body(g, carry):
            for sub in range(2):
                c = g * 2 + sub
                buf = rows[sub]

                @pl.when(c >= 2)
                def _():
                    drain_write(buf)

                expand_chunk(c, buf)
                start_write(c, buf)
            return carry

        lax.fori_loop(0, n_ch // 2, g_body, 0)
        drain_write(rows0)
        drain_write(rows1)

    return body(pos2d, lut, emb_flat)


def kernel(pair_tensor, lookup_table, embedding):
    Bo, N, _ = pair_tensor.shape
    D = embedding.shape[1]
    W = lookup_table.shape[1]
    pos2d = pair_tensor[..., 0] * W + pair_tensor[..., 1]
    lut = lookup_table.reshape(W * W)
    out = _sc_lookup(pos2d, lut, embedding.reshape(D * D), N, D, W)
    return out.reshape(Bo, N, D)
